# Initial kernel scaffold; baseline (speedup 1.0000x reference)
import jax
import jax.numpy as jnp
from jax.experimental import pallas as pl


def kernel(x, edge_index, W1_l, W1_r, b1, W2_l, W2_r, b2):
    raise NotImplementedError("write your pallas kernel here")


# speedup vs baseline: 5.2856x; 5.2856x over previous
import functools

import jax
import jax.numpy as jnp
from jax import lax
from jax.experimental import pallas as pl
from jax.experimental.pallas import tpu as pltpu
from jax.experimental.pallas import tpu_sc as plsc

N = 10000
NP = 10240
E = 320000
NC = 2
NS = 16
NW = NC * NS
EPW = E // NW
B = 80
NCHUNK = EPW // B
RPT = NP // NS
CW = 16

_f32 = jnp.float32


def _agg_body(D, with_cnt, *refs):
    if with_cnt:
        (feat, src, dst, ones_h, zf, zc,
         accp_out, cntp_out,
         src_v, dst_v, rows_v, ones_v, acc, cacc, sem) = refs
    else:
        (feat, src, dst, zf,
         accp_out,
         src_v, dst_v, rows_v, acc, sem) = refs

    cid = lax.axis_index("c")
    sid = lax.axis_index("s")
    wid = sid * NC + cid
    r0 = sid * RPT

    pltpu.sync_copy(zf.at[pl.ds(r0, RPT)], acc.at[pl.ds(r0, RPT)])
    if with_cnt:
        pltpu.sync_copy(zc.at[pl.ds(r0, RPT)], cacc.at[pl.ds(r0, RPT)])
        pltpu.sync_copy(ones_h, ones_v)
    plsc.subcore_barrier()

    e0 = wid * EPW

    def chunk(j, carry):
        base = e0 + j * B
        pltpu.sync_copy(src.at[pl.ds(base, B)], src_v)
        pltpu.sync_copy(dst.at[pl.ds(base, B)], dst_v)
        pltpu.async_copy(feat.at[src_v], rows_v, sem).wait()
        pltpu.sync_copy(rows_v, acc.at[dst_v], add=True)
        if with_cnt:
            pltpu.sync_copy(ones_v, cacc.at[dst_v], add=True)
        return carry

    lax.fori_loop(0, NCHUNK, chunk, 0)
    plsc.subcore_barrier()

    pltpu.sync_copy(acc.at[pl.ds(r0, RPT)], accp_out.at[cid, pl.ds(r0, RPT)])
    if with_cnt:
        pltpu.sync_copy(cacc.at[pl.ds(r0, RPT)],
                        cntp_out.at[cid, pl.ds(r0, RPT)])


def _make_agg(D, with_cnt):
    mesh = plsc.VectorSubcoreMesh(core_axis_name="c", subcore_axis_name="s")
    out_type = [jax.ShapeDtypeStruct((NC, NP, D), _f32)]
    if with_cnt:
        out_type.append(jax.ShapeDtypeStruct((NC, NP, CW), _f32))
    scratch = [
        pltpu.VMEM((B,), jnp.int32),
        pltpu.VMEM((B,), jnp.int32),
        pltpu.VMEM((B, D), _f32),
    ]
    if with_cnt:
        scratch.append(pltpu.VMEM((B, CW), _f32))
    scratch.append(pltpu.VMEM_SHARED((NP, D), _f32))
    if with_cnt:
        scratch.append(pltpu.VMEM_SHARED((NP, CW), _f32))
    scratch.append(pltpu.SemaphoreType.DMA)
    return pl.kernel(
        functools.partial(_agg_body, D, with_cnt),
        out_type=out_type,
        mesh=mesh,
        scratch_types=scratch,
        compiler_params=pltpu.CompilerParams(use_tc_tiling_on_sc=False),
    )


def _tc1_body(aggp, cntp, x, w1l, w1r, b1, w2lp, w2rp, b2p, y2_out, z_out):
    agg = aggp[0] + aggp[1]
    cnt = cntp[0][:, 0:1] + cntp[1][:, 0:1]
    mean = agg * (1.0 / jnp.maximum(cnt, 1.0))
    h = jnp.dot(mean, w1l[...], preferred_element_type=_f32)
    h += jnp.dot(x[...], w1r[...], preferred_element_type=_f32)
    h = jnp.maximum(h + b1[0], 0.0)
    y2_out[...] = jnp.dot(h, w2lp[...], preferred_element_type=_f32)
    z_out[...] = jnp.dot(h, w2rp[...], preferred_element_type=_f32) + b2p[0]


def _tc2_body(agg2p, cntp, z, out_ref):
    agg2 = agg2p[0] + agg2p[1]
    cnt = cntp[0][:, 0:1] + cntp[1][:, 0:1]
    o = z[...] + agg2 * (1.0 / jnp.maximum(cnt, 1.0))
    m = jnp.max(o, axis=-1, keepdims=True)
    e = jnp.exp(o - m)
    s = jnp.sum(e, axis=-1, keepdims=True)
    out_ref[...] = o - m - jnp.log(s)


def kernel(x, edge_index, W1_l, W1_r, b1, W2_l, W2_r, b2):
    D = x.shape[1]
    DO = W2_l.shape[1]
    DP = 64
    src = edge_index[0]
    dst = edge_index[1]

    ones_rows = jnp.ones((B, CW), _f32)
    zf128 = jnp.zeros((NP, D), _f32)
    zc = jnp.zeros((NP, CW), _f32)
    zf64 = jnp.zeros((NP, DP), _f32)

    agg1p, cntp = _make_agg(D, True)(x, src, dst, ones_rows, zf128, zc)

    w2lp = jnp.pad(W2_l, ((0, 0), (0, DP - DO)))
    w2rp = jnp.pad(W2_r, ((0, 0), (0, DP - DO)))
    b2p = jnp.pad(b2, (0, DP - DO), constant_values=-1e30).reshape(1, DP)
    b1r = b1.reshape(1, D)

    bn = 1000
    grid = (N // bn,)
    y2, z = pl.pallas_call(
        _tc1_body,
        grid=grid,
        in_specs=[
            pl.BlockSpec((NC, bn, D), lambda i: (0, i, 0)),
            pl.BlockSpec((NC, bn, CW), lambda i: (0, i, 0)),
            pl.BlockSpec((bn, D), lambda i: (i, 0)),
            pl.BlockSpec((D, D), lambda i: (0, 0)),
            pl.BlockSpec((D, D), lambda i: (0, 0)),
            pl.BlockSpec((1, D), lambda i: (0, 0)),
            pl.BlockSpec((D, DP), lambda i: (0, 0)),
            pl.BlockSpec((D, DP), lambda i: (0, 0)),
            pl.BlockSpec((1, DP), lambda i: (0, 0)),
        ],
        out_specs=[
            pl.BlockSpec((bn, DP), lambda i: (i, 0)),
            pl.BlockSpec((bn, DP), lambda i: (i, 0)),
        ],
        out_shape=[
            jax.ShapeDtypeStruct((N, DP), _f32),
            jax.ShapeDtypeStruct((N, DP), _f32),
        ],
    )(agg1p, cntp, x, W1_l, W1_r, b1r, w2lp, w2rp, b2p)

    (agg2p,) = _make_agg(DP, False)(y2, src, dst, zf64)

    out64 = pl.pallas_call(
        _tc2_body,
        grid=grid,
        in_specs=[
            pl.BlockSpec((NC, bn, DP), lambda i: (0, i, 0)),
            pl.BlockSpec((NC, bn, CW), lambda i: (0, i, 0)),
            pl.BlockSpec((bn, DP), lambda i: (i, 0)),
        ],
        out_specs=pl.BlockSpec((bn, DP), lambda i: (i, 0)),
        out_shape=jax.ShapeDtypeStruct((N, DP), _f32),
    )(agg2p, cntp, z)

    return out64[:, :DO]
